# Initial kernel scaffold; baseline (speedup 1.0000x reference)
import jax
import jax.numpy as jnp
from jax.experimental import pallas as pl


def kernel(positions, lattice, numbers):
    raise NotImplementedError("write your pallas kernel here")


# speedup vs baseline: 1.4822x; 1.4822x over previous
import jax
import jax.numpy as jnp
from jax.experimental import pallas as pl
from jax.experimental.pallas import tpu as pltpu

_CUTOFF = 5.0
_N = 1024
_TI = 256


def _dist_kernel(shifts_ref, pos_ref, post_ref, dist_ref, mask_ref):
    s = pl.program_id(0)
    pxi = pos_ref[:, 0:1]
    pyi = pos_ref[:, 1:2]
    pzi = pos_ref[:, 2:3]
    pxj = post_ref[0:1, :]
    pyj = post_ref[1:2, :]
    pzj = post_ref[2:3, :]
    dx = (pxj + shifts_ref[s, 0]) - pxi
    dy = (pyj + shifts_ref[s, 1]) - pyi
    dz = (pzj + shifts_ref[s, 2]) - pzi
    d = jnp.sqrt(dx * dx + dy * dy + dz * dz + 1e-12)
    dist_ref[0, :, :] = d
    mask_ref[0, :, :] = (d < _CUTOFF) & (d > 1e-6)


def kernel(positions, lattice, numbers):
    shifts_frac = jnp.asarray(
        [[i, j, k] for i in (-1, 0, 1) for j in (-1, 0, 1) for k in (-1, 0, 1)],
        dtype=jnp.float32,
    )
    shifts_cart = shifts_frac @ lattice
    pos = positions @ lattice
    post = pos.T

    n = pos.shape[0]
    nblk = n // _TI
    grid = (27, nblk)

    dist, mask = pl.pallas_call(
        _dist_kernel,
        grid=grid,
        in_specs=[
            pl.BlockSpec(memory_space=pltpu.SMEM),
            pl.BlockSpec((_TI, 3), lambda s, i: (i, 0)),
            pl.BlockSpec((3, n), lambda s, i: (0, 0)),
        ],
        out_specs=[
            pl.BlockSpec((1, _TI, n), lambda s, i: (s, i, 0)),
            pl.BlockSpec((1, _TI, n), lambda s, i: (s, i, 0)),
        ],
        out_shape=[
            jax.ShapeDtypeStruct((27, n, n), jnp.float32),
            jax.ShapeDtypeStruct((27, n, n), jnp.bool_),
        ],
    )(shifts_cart, pos, post)
    return dist, mask


# speedup vs baseline: 1.5335x; 1.0346x over previous
import jax
import jax.numpy as jnp
from jax.experimental import pallas as pl
from jax.experimental.pallas import tpu as pltpu

_CUTOFF = 5.0
_N = 1024
_TI = 256


def _dist_kernel(shifts_ref, pos_ref, post_ref, dist_ref, mask_ref):
    s = pl.program_id(0)
    pxi = pos_ref[:, 0:1]
    pyi = pos_ref[:, 1:2]
    pzi = pos_ref[:, 2:3]
    pxj = post_ref[0:1, :]
    pyj = post_ref[1:2, :]
    pzj = post_ref[2:3, :]
    dx = (pxj + shifts_ref[s, 0]) - pxi
    dy = (pyj + shifts_ref[s, 1]) - pyi
    dz = (pzj + shifts_ref[s, 2]) - pzi
    y = dx * dx + dy * dy + dz * dz + 1e-12
    dist_ref[0, :, :] = y * jax.lax.rsqrt(y)
    mask_ref[0, :, :] = (y < 24.999998) & (y > 1.0000001e-12)


def kernel(positions, lattice, numbers):
    shifts_frac = jnp.asarray(
        [[i, j, k] for i in (-1, 0, 1) for j in (-1, 0, 1) for k in (-1, 0, 1)],
        dtype=jnp.float32,
    )
    shifts_cart = shifts_frac @ lattice
    pos = positions @ lattice
    post = pos.T

    n = pos.shape[0]
    nblk = n // _TI
    grid = (27, nblk)

    dist, mask = pl.pallas_call(
        _dist_kernel,
        grid=grid,
        in_specs=[
            pl.BlockSpec(memory_space=pltpu.SMEM),
            pl.BlockSpec((_TI, 3), lambda s, i: (i, 0)),
            pl.BlockSpec((3, n), lambda s, i: (0, 0)),
        ],
        out_specs=[
            pl.BlockSpec((1, _TI, n), lambda s, i: (s, i, 0)),
            pl.BlockSpec((1, _TI, n), lambda s, i: (s, i, 0)),
        ],
        out_shape=[
            jax.ShapeDtypeStruct((27, n, n), jnp.float32),
            jax.ShapeDtypeStruct((27, n, n), jnp.bool_),
        ],
    )(shifts_cart, pos, post)
    return dist, mask


# speedup vs baseline: 1.8416x; 1.2009x over previous
import jax
import jax.numpy as jnp
from jax.experimental import pallas as pl
from jax.experimental.pallas import tpu as pltpu

_CUTOFF = 5.0
_N = 1024
_TI = 512


def _dist_kernel(shifts_ref, pos_ref, post_ref, dist_ref, mask_ref):
    s = pl.program_id(0)
    pxi = pos_ref[:, 0:1]
    pyi = pos_ref[:, 1:2]
    pzi = pos_ref[:, 2:3]
    pxj = post_ref[0:1, :]
    pyj = post_ref[1:2, :]
    pzj = post_ref[2:3, :]
    dx = (pxj + shifts_ref[s, 0]) - pxi
    dy = (pyj + shifts_ref[s, 1]) - pyi
    dz = (pzj + shifts_ref[s, 2]) - pzi
    y = dx * dx + dy * dy + dz * dz + 1e-12
    dist_ref[0, :, :] = y * jax.lax.rsqrt(y)
    mask_ref[0, :, :] = (y < 24.999998) & (y > 1.0000001e-12)


def kernel(positions, lattice, numbers):
    shifts_frac = jnp.asarray(
        [[i, j, k] for i in (-1, 0, 1) for j in (-1, 0, 1) for k in (-1, 0, 1)],
        dtype=jnp.float32,
    )
    shifts_cart = shifts_frac @ lattice
    pos = positions @ lattice
    post = pos.T

    n = pos.shape[0]
    nblk = n // _TI
    grid = (27, nblk)

    dist, mask = pl.pallas_call(
        _dist_kernel,
        grid=grid,
        in_specs=[
            pl.BlockSpec(memory_space=pltpu.SMEM),
            pl.BlockSpec((_TI, 3), lambda s, i: (i, 0)),
            pl.BlockSpec((3, n), lambda s, i: (0, 0)),
        ],
        out_specs=[
            pl.BlockSpec((1, _TI, n), lambda s, i: (s, i, 0)),
            pl.BlockSpec((1, _TI, n), lambda s, i: (s, i, 0)),
        ],
        out_shape=[
            jax.ShapeDtypeStruct((27, n, n), jnp.float32),
            jax.ShapeDtypeStruct((27, n, n), jnp.bool_),
        ],
    )(shifts_cart, pos, post)
    return dist, mask


# speedup vs baseline: 2.6735x; 1.4517x over previous
import jax
import jax.numpy as jnp
from jax.experimental import pallas as pl
from jax.experimental.pallas import tpu as pltpu

_N = 1024
_TC = 256


def _dist_kernel(shifts_ref, pos_ref, post_ref, dist_ref, mask_ref):
    s = pl.program_id(0)
    pxj = post_ref[0:1, :]
    pyj = post_ref[1:2, :]
    pzj = post_ref[2:3, :]
    cxj = pxj + shifts_ref[s, 0]
    cyj = pyj + shifts_ref[s, 1]
    czj = pzj + shifts_ref[s, 2]
    for r in range(_N // _TC):
        rows = pl.ds(r * _TC, _TC)
        dx = cxj - pos_ref[rows, 0:1]
        dy = cyj - pos_ref[rows, 1:2]
        dz = czj - pos_ref[rows, 2:3]
        y = dx * dx + dy * dy + dz * dz + 1e-12
        dist_ref[0, rows, :] = y * jax.lax.rsqrt(y)
        mask_ref[0, rows, :] = ((y < 24.999998) & (y > 1.0000001e-12)).astype(jnp.int8)


def kernel(positions, lattice, numbers):
    shifts_frac = jnp.asarray(
        [[i, j, k] for i in (-1, 0, 1) for j in (-1, 0, 1) for k in (-1, 0, 1)],
        dtype=jnp.float32,
    )
    shifts_cart = shifts_frac @ lattice
    pos = positions @ lattice
    post = pos.T

    n = pos.shape[0]

    dist, mask = pl.pallas_call(
        _dist_kernel,
        grid=(27,),
        in_specs=[
            pl.BlockSpec(memory_space=pltpu.SMEM),
            pl.BlockSpec((n, 3), lambda s: (0, 0)),
            pl.BlockSpec((3, n), lambda s: (0, 0)),
        ],
        out_specs=[
            pl.BlockSpec((1, n, n), lambda s: (s, 0, 0)),
            pl.BlockSpec((1, n, n), lambda s: (s, 0, 0)),
        ],
        out_shape=[
            jax.ShapeDtypeStruct((27, n, n), jnp.float32),
            jax.ShapeDtypeStruct((27, n, n), jnp.int8),
        ],
    )(shifts_cart, pos, post)
    return dist, mask.astype(jnp.bool_)
